# baseline (speedup 1.0000x reference)
import functools

import jax
import jax.numpy as jnp
from jax import lax
from jax.experimental import pallas as pl
from jax.experimental.pallas import tpu as pltpu
from jax.experimental.pallas import tpu_sc as plsc

MAX_LANG_VOCAB_IDX = 100000
N_LANGS = 4096
EMBED_DIM = 64
BATCH = 16384
HIST = 200

N_TOKENS = BATCH * HIST
NW = 32
CHUNK = 128
NBT = BATCH // CHUNK
BT_PER_W = NBT // NW
XB = CHUNK * HIST
LMAP_PAD = 100352
LMAP_W = LMAP_PAD // 2
ET = EMBED_DIM // 8


def _sc_kernel(x_hbm, lmap_hbm, cap_hbm, w_hbm, out_hbm,
               lmap_v, xb_v, idx_v, rows_v, t_v, cap_v, gsems, wsems):
    wid = lax.axis_index("s") * 2 + lax.axis_index("c")

    pltpu.sync_copy(lmap_hbm, lmap_v)
    pltpu.sync_copy(cap_hbm, cap_v)
    cap = cap_v[...]

    lane = lax.iota(jnp.int32, 16)

    def remap(h, p):
        for j in range(CHUNK // 16):
            xv = plsc.load_gather(xb_v, [(lane + j * 16) * HIST + h])
            xc = jnp.minimum(xv, cap)
            word = plsc.load_gather(lmap_v, [lax.shift_right_logical(xc, 1)])
            sh = jnp.left_shift(jnp.bitwise_and(xc, 1), 4)
            y = jnp.bitwise_and(lax.shift_right_logical(word, sh), 0xFFFF)
            idx_v[p, pl.ds(j * 16, 16)] = y

    def transpose(q):
        def tb(et, carry):
            for e8 in range(8):
                ev = jnp.full((16,), et * 8 + e8, jnp.int32)
                for j in range(CHUNK // 16):
                    v = plsc.load_gather(rows_v.at[q], [lane + j * 16, ev])
                    t_v[q, et, pl.ds(e8 * CHUNK + j * 16, 16)] = v
            return carry

        lax.fori_loop(0, ET, tb, 0)

    def fire_gather(p):
        pltpu.async_copy(w_hbm.at[idx_v.at[p]], rows_v.at[p], gsems[p])

    def drain_gather(p):
        pltpu.make_async_copy(
            w_hbm.at[idx_v.at[p]], rows_v.at[p], gsems[p]
        ).wait()

    def out_block(h, btg):
        return out_hbm.at[h, :, btg]

    def fire_write(h, q, btg):
        pltpu.async_copy(t_v.at[q], out_block(h, btg), wsems[q])

    def drain_write(h, q, btg):
        pltpu.make_async_copy(t_v.at[q], out_block(h, btg), wsems[q]).wait()

    def bt_body(k, carry0):
        btg = wid * BT_PER_W + k
        pltpu.sync_copy(x_hbm.at[pl.ds(btg * XB, XB)], xb_v)

        def body(h2, carry):
            for p in range(2):
                h = 2 * h2 + p
                q = 1 - p
                remap(h, p)

                @pl.when(h2 > 0)
                def _():
                    drain_write(h - 2, p, btg)

                fire_gather(p)

                def retire():
                    drain_gather(q)
                    transpose(q)
                    fire_write(h - 1, q, btg)

                if p == 1:
                    retire()
                else:
                    pl.when(h2 > 0)(retire)
            return carry

        lax.fori_loop(0, HIST // 2, body, 0)

        drain_gather(1)
        transpose(1)
        fire_write(HIST - 1, 1, btg)
        drain_write(HIST - 2, 0, btg)
        drain_write(HIST - 1, 1, btg)
        return carry0

    lax.fori_loop(0, BT_PER_W, bt_body, 0)


@jax.jit
def _run(x_flat, lmap_packed, cap, weight):
    mesh = plsc.VectorSubcoreMesh(core_axis_name="c", subcore_axis_name="s")
    f = functools.partial(
        pl.kernel,
        out_type=jax.ShapeDtypeStruct((HIST, ET, NBT, 8 * CHUNK), jnp.float32),
        mesh=mesh,
        compiler_params=pltpu.CompilerParams(
            needs_layout_passes=False, use_tc_tiling_on_sc=False
        ),
        scratch_types=[
            pltpu.VMEM((LMAP_W,), jnp.int32),
            pltpu.VMEM((XB,), jnp.int32),
            pltpu.VMEM((2, CHUNK), jnp.int32),
            pltpu.VMEM((2, CHUNK, EMBED_DIM), jnp.float32),
            pltpu.VMEM((2, ET, 8 * CHUNK), jnp.float32),
            pltpu.VMEM((16,), jnp.int32),
            [pltpu.SemaphoreType.DMA] * 2,
            [pltpu.SemaphoreType.DMA] * 2,
        ],
    )(_sc_kernel)
    return f(x_flat, lmap_packed, cap, weight)


def kernel(x, lang_map, max_lang_vocab_idx, weight):
    x_flat = x.reshape(-1)
    lmap_pad = jnp.zeros((LMAP_PAD,), jnp.int32).at[: lang_map.shape[0]].set(lang_map)
    lmap_packed = lmap_pad[0::2] | jnp.left_shift(lmap_pad[1::2], 16)
    cap_vec = jnp.broadcast_to(max_lang_vocab_idx.astype(jnp.int32), (16,))
    p5 = _run(x_flat, lmap_packed, cap_vec, weight)
    p5 = p5.reshape(HIST, ET, NBT, 8, CHUNK)
    return p5.transpose(2, 4, 0, 1, 3).reshape(BATCH, HIST, EMBED_DIM)


# speedup vs baseline: 3.3777x; 3.3777x over previous
import functools

import jax
import jax.numpy as jnp
from jax import lax
from jax.experimental import pallas as pl
from jax.experimental.pallas import tpu as pltpu
from jax.experimental.pallas import tpu_sc as plsc

MAX_LANG_VOCAB_IDX = 100000
N_LANGS = 4096
EMBED_DIM = 64
BATCH = 16384
HIST = 200

N_TOKENS = BATCH * HIST
NW = 32
TOK_PER_W = N_TOKENS // NW
CHUNK = 128
NGRP = 2
GCHUNK = 4
STEP = CHUNK * GCHUNK * NGRP
N_STEPS = TOK_PER_W // STEP
LMAP_PAD = 100352
LMAP_W = LMAP_PAD // 2


HALF = CHUNK * GCHUNK
N_HALVES = TOK_PER_W // HALF


def _sc_kernel(x_hbm, lmap_hbm, cap_hbm, w_hbm, out_hbm,
               lmap_v, x_v, idx_v, rows_v, cap_v, xsems, gsems, wsems):
    wid = lax.axis_index("s") * 2 + lax.axis_index("c")
    base_w = wid * TOK_PER_W

    pltpu.sync_copy(lmap_hbm, lmap_v)
    pltpu.sync_copy(cap_hbm, cap_v)
    cap = cap_v[...]

    def x_block(h, p):
        base = jnp.minimum(base_w + h * HALF, N_TOKENS - HALF)
        return x_hbm.at[pl.ds(base, HALF)], x_v.at[p]

    def out_slice(h, u):
        off = base_w + h * HALF + u * CHUNK
        return out_hbm.at[pl.ds(off, CHUNK), pl.ds(0, EMBED_DIM)]

    def drain_gathers(p):
        for u in range(GCHUNK):
            pltpu.make_async_copy(
                w_hbm.at[idx_v.at[p, u]], rows_v.at[p, u], gsems[p]
            ).wait()

    def fire_writes(h, p):
        for u in range(GCHUNK):
            pltpu.async_copy(rows_v.at[p, u], out_slice(h, u), wsems[p])

    def drain_writes(h, p):
        for u in range(GCHUNK):
            pltpu.make_async_copy(
                rows_v.at[p, u], out_slice(h, u), wsems[p]
            ).wait()

    pltpu.async_copy(*x_block(0, 0), xsems[0])
    pltpu.async_copy(*x_block(1, 1), xsems[1])

    def body(s, carry):
        for p in range(NGRP):
            h = 2 * s + p
            @pl.when(h > 0)
            def _():
                drain_gathers(1 - p)
                fire_writes(h - 1, 1 - p)

            pltpu.make_async_copy(*x_block(h, p), xsems[p]).wait()
            for u in range(GCHUNK):
                for j in range(CHUNK // 16):
                    xv = x_v[p, pl.ds(u * CHUNK + j * 16, 16)]
                    xc = jnp.minimum(xv, cap)
                    word = plsc.load_gather(
                        lmap_v, [lax.shift_right_logical(xc, 1)]
                    )
                    sh = jnp.left_shift(jnp.bitwise_and(xc, 1), 4)
                    y = jnp.bitwise_and(
                        lax.shift_right_logical(word, sh), 0xFFFF
                    )
                    idx_v[p, u, pl.ds(j * 16, 16)] = y

            @pl.when(s > 0)
            def _():
                drain_writes(h - 2, p)

            for u in range(GCHUNK):
                pltpu.async_copy(
                    w_hbm.at[idx_v.at[p, u]], rows_v.at[p, u], gsems[p]
                )
            pltpu.async_copy(*x_block(h + 2, p), xsems[p])
        return carry

    lax.fori_loop(0, N_HALVES // 2, body, 0)

    last = N_HALVES - 1
    drain_gathers(1)
    fire_writes(last, 1)
    drain_writes(last - 1, 0)
    drain_writes(last, 1)
    pltpu.make_async_copy(*x_block(last + 1, 0), xsems[0]).wait()
    pltpu.make_async_copy(*x_block(last + 2, 1), xsems[1]).wait()


@jax.jit
def _run(x_flat, lmap_packed, cap, weight):
    mesh = plsc.VectorSubcoreMesh(core_axis_name="c", subcore_axis_name="s")
    f = functools.partial(
        pl.kernel,
        out_type=jax.ShapeDtypeStruct((N_TOKENS, 128), jnp.float32),
        mesh=mesh,
        compiler_params=pltpu.CompilerParams(
            needs_layout_passes=False, use_tc_tiling_on_sc=False
        ),
        scratch_types=[
            pltpu.VMEM((LMAP_W,), jnp.int32),
            pltpu.VMEM((NGRP, HALF), jnp.int32),
            pltpu.VMEM((NGRP, GCHUNK, CHUNK), jnp.int32),
            pltpu.VMEM((NGRP, GCHUNK, CHUNK, EMBED_DIM), jnp.float32),
            pltpu.VMEM((16,), jnp.int32),
            [pltpu.SemaphoreType.DMA] * NGRP,
            [pltpu.SemaphoreType.DMA] * NGRP,
            [pltpu.SemaphoreType.DMA] * NGRP,
        ],
    )(_sc_kernel)
    return f(x_flat, lmap_packed, cap, weight)


def kernel(x, lang_map, max_lang_vocab_idx, weight):
    x_flat = x.reshape(-1)
    lmap_pad = jnp.zeros((LMAP_PAD,), jnp.int32).at[: lang_map.shape[0]].set(lang_map)
    lmap_packed = lmap_pad[0::2] | jnp.left_shift(lmap_pad[1::2], 16)
    cap_vec = jnp.broadcast_to(max_lang_vocab_idx.astype(jnp.int32), (16,))
    out = _run(x_flat, lmap_packed, cap_vec, weight)
    return out[:, :EMBED_DIM].reshape(BATCH, HIST, EMBED_DIM)
